# Initial kernel scaffold; baseline (speedup 1.0000x reference)
import jax
import jax.numpy as jnp
from jax.experimental import pallas as pl


def kernel(LUT, x):
    raise NotImplementedError("write your pallas kernel here")


# speedup vs baseline: 1288.0912x; 1288.0912x over previous
import functools

import jax
import jax.numpy as jnp
from jax import lax
from jax.experimental import pallas as pl
from jax.experimental.pallas import tpu as pltpu
from jax.experimental.pallas import tpu_sc as plsc

DIM = 33
LANES = 16


def _make_sc_call(n_rows, n_pix_per_batch, chunk):
    info = plsc.get_sparse_core_info()
    NC, NS = info.num_cores, info.num_subcores
    NW = NC * NS
    n_batch = n_rows // 3
    tiles_per_batch = NW // n_batch
    pix_per_tile = n_pix_per_batch // tiles_per_batch
    n_chunks = pix_per_tile // chunk
    dim2 = DIM * DIM

    mesh = plsc.VectorSubcoreMesh(core_axis_name="c", subcore_axis_name="s")

    @functools.partial(
        pl.kernel,
        mesh=mesh,
        out_type=jax.ShapeDtypeStruct((n_rows, n_pix_per_batch), jnp.float32),
        compiler_params=pltpu.CompilerParams(needs_layout_passes=False),
        scratch_types=[
            pltpu.VMEM((3 * DIM * DIM * DIM,), jnp.float32),
            pltpu.VMEM((chunk,), jnp.float32),
            pltpu.VMEM((chunk,), jnp.float32),
            pltpu.VMEM((chunk,), jnp.float32),
            pltpu.VMEM((chunk,), jnp.float32),
            pltpu.VMEM((chunk,), jnp.float32),
            pltpu.VMEM((chunk,), jnp.float32),
        ],
    )
    def call(lut_hbm, x_hbm, out_hbm, lut_v, in_r, in_g, in_b,
             out_0, out_1, out_2):
        cid = lax.axis_index("c")
        sid = lax.axis_index("s")
        wid = sid * NC + cid
        batch = wid // tiles_per_batch
        quarter = wid % tiles_per_batch
        row0 = batch * 3

        pltpu.sync_copy(lut_hbm, lut_v)

        cone = jnp.full((LANES,), 1, jnp.int32)
        vdim = jnp.full((LANES,), DIM, jnp.int32)
        vtab = jnp.full((LANES,), DIM * DIM * DIM, jnp.int32)
        vdim2 = jnp.full((LANES,), dim2, jnp.int32)
        vmaxid = jnp.full((LANES,), DIM - 2, jnp.int32)
        vzero_i = jnp.zeros((LANES,), jnp.int32)
        vscale = jnp.full((LANES,), float(DIM - 1), jnp.float32)
        vone = jnp.full((LANES,), 1.0, jnp.float32)

        def chunk_body(k, _):
            off = quarter * pix_per_tile + k * chunk
            pltpu.sync_copy(x_hbm.at[row0 + 0, pl.ds(off, chunk)], in_r)
            pltpu.sync_copy(x_hbm.at[row0 + 1, pl.ds(off, chunk)], in_g)
            pltpu.sync_copy(x_hbm.at[row0 + 2, pl.ds(off, chunk)], in_b)

            def pix_body(i, _):
                sl = pl.ds(i * LANES, LANES)
                rs = in_r[sl] * vscale
                gs = in_g[sl] * vscale
                bs = in_b[sl] * vscale
                rid = lax.max(lax.min(rs.astype(jnp.int32), vmaxid), vzero_i)
                gid = lax.max(lax.min(gs.astype(jnp.int32), vmaxid), vzero_i)
                bid = lax.max(lax.min(bs.astype(jnp.int32), vmaxid), vzero_i)
                rd = rs - rid.astype(jnp.float32)
                gd = gs - gid.astype(jnp.float32)
                bd = bs - bid.astype(jnp.float32)
                id000 = rid + gid * vdim + bid * vdim2
                id100 = id000 + cone
                id010 = id000 + vdim
                id110 = id010 + cone
                id001 = id000 + vdim2
                id101 = id001 + cone
                id011 = id001 + vdim
                id111 = id011 + cone
                rd1 = vone - rd
                gd1 = vone - gd
                bd1 = vone - bd
                w00 = rd1 * gd1
                w10 = rd * gd1
                w01 = rd1 * gd
                w11 = rd * gd
                w000 = w00 * bd1
                w100 = w10 * bd1
                w010 = w01 * bd1
                w110 = w11 * bd1
                w001 = w00 * bd
                w101 = w10 * bd
                w011 = w01 * bd
                w111 = w11 * bd

                for out_ref in (out_0, out_1, out_2):
                    acc = w000 * plsc.load_gather(lut_v, [id000])
                    acc = acc + w100 * plsc.load_gather(lut_v, [id100])
                    acc = acc + w010 * plsc.load_gather(lut_v, [id010])
                    acc = acc + w110 * plsc.load_gather(lut_v, [id110])
                    acc = acc + w001 * plsc.load_gather(lut_v, [id001])
                    acc = acc + w101 * plsc.load_gather(lut_v, [id101])
                    acc = acc + w011 * plsc.load_gather(lut_v, [id011])
                    acc = acc + w111 * plsc.load_gather(lut_v, [id111])
                    out_ref[sl] = acc
                    id000 = id000 + vtab
                    id100 = id100 + vtab
                    id010 = id010 + vtab
                    id110 = id110 + vtab
                    id001 = id001 + vtab
                    id101 = id101 + vtab
                    id011 = id011 + vtab
                    id111 = id111 + vtab
                return 0

            lax.fori_loop(0, chunk // LANES, pix_body, 0)

            pltpu.sync_copy(out_0, out_hbm.at[row0 + 0, pl.ds(off, chunk)])
            pltpu.sync_copy(out_1, out_hbm.at[row0 + 1, pl.ds(off, chunk)])
            pltpu.sync_copy(out_2, out_hbm.at[row0 + 2, pl.ds(off, chunk)])
            return 0

        lax.fori_loop(0, n_chunks, chunk_body, 0)

    return call


def kernel(LUT, x):
    B, C, H, W = x.shape
    n_pix = H * W
    xr = x.reshape(B * C, n_pix)
    lut_flat = LUT.reshape(3 * DIM * DIM * DIM)
    call = _make_sc_call(B * C, n_pix, 2048)
    out = call(lut_flat, xr)
    return out.reshape(B, C, H, W)


# speedup vs baseline: 1608.9943x; 1.2491x over previous
import functools

import jax
import jax.numpy as jnp
from jax import lax
from jax.experimental import pallas as pl
from jax.experimental.pallas import tpu as pltpu
from jax.experimental.pallas import tpu_sc as plsc

DIM = 33
LANES = 16
CHUNK = 1024


def _make_sc_call(n_rows, n_pix_per_batch):
    info = plsc.get_sparse_core_info()
    NC, NS = info.num_cores, info.num_subcores
    NW = NC * NS
    n_batch = n_rows // 3
    tiles_per_batch = NW // n_batch
    pix_per_tile = n_pix_per_batch // tiles_per_batch
    n_chunks = pix_per_tile // CHUNK
    dim2 = DIM * DIM
    n_tab = DIM * DIM * DIM

    mesh = plsc.VectorSubcoreMesh(core_axis_name="c", subcore_axis_name="s")

    @functools.partial(
        pl.kernel,
        mesh=mesh,
        out_type=jax.ShapeDtypeStruct((n_rows, n_pix_per_batch), jnp.float32),
        compiler_params=pltpu.CompilerParams(needs_layout_passes=False),
        scratch_types=[
            pltpu.VMEM((n_tab,), jnp.float32),
            pltpu.VMEM((n_tab,), jnp.float32),
            pltpu.VMEM((n_tab,), jnp.float32),
        ] + [pltpu.VMEM((CHUNK,), jnp.float32)] * 12 + [
            pltpu.SemaphoreType.DMA,
            pltpu.SemaphoreType.DMA,
            pltpu.SemaphoreType.DMA,
            pltpu.SemaphoreType.DMA,
        ],
    )
    def call(lut0_hbm, lut1_hbm, lut2_hbm, x_hbm, out_hbm,
             lut0, lut1, lut2,
             in0r, in0g, in0b, in1r, in1g, in1b,
             out0r, out0g, out0b, out1r, out1g, out1b,
             sin0, sin1, sout0, sout1):
        in0 = (in0r, in0g, in0b)
        in1 = (in1r, in1g, in1b)
        out0 = (out0r, out0g, out0b)
        out1 = (out1r, out1g, out1b)
        cid = lax.axis_index("c")
        sid = lax.axis_index("s")
        wid = sid * NC + cid
        batch = wid // tiles_per_batch
        quarter = wid % tiles_per_batch
        row0 = batch * 3
        base = quarter * pix_per_tile

        pltpu.sync_copy(lut0_hbm, lut0)
        pltpu.sync_copy(lut1_hbm, lut1)
        pltpu.sync_copy(lut2_hbm, lut2)

        cone = jnp.full((LANES,), 1, jnp.int32)
        vdim = jnp.full((LANES,), DIM, jnp.int32)
        vdim2 = jnp.full((LANES,), dim2, jnp.int32)
        vmaxid = jnp.full((LANES,), DIM - 2, jnp.int32)
        vscale = jnp.full((LANES,), float(DIM - 1), jnp.float32)
        vone = jnp.full((LANES,), 1.0, jnp.float32)

        def in_copies(k, buf, sem):
            off = base + k * CHUNK
            return [
                pltpu.make_async_copy(
                    x_hbm.at[row0 + c, pl.ds(off, CHUNK)], buf[c], sem)
                for c in range(3)
            ]

        def out_copies(k, buf, sem):
            off = base + k * CHUNK
            return [
                pltpu.make_async_copy(
                    buf[c], out_hbm.at[row0 + c, pl.ds(off, CHUNK)], sem)
                for c in range(3)
            ]

        def compute(in_v, out_v):
            def pix_body(i, _):
                sl = pl.ds(i * LANES, LANES)
                rs = in_v[0][sl] * vscale
                gs = in_v[1][sl] * vscale
                bs = in_v[2][sl] * vscale
                rid = lax.min(rs.astype(jnp.int32), vmaxid)
                gid = lax.min(gs.astype(jnp.int32), vmaxid)
                bid = lax.min(bs.astype(jnp.int32), vmaxid)
                rd = rs - rid.astype(jnp.float32)
                gd = gs - gid.astype(jnp.float32)
                bd = bs - bid.astype(jnp.float32)
                id000 = rid + gid * vdim + bid * vdim2
                id100 = id000 + cone
                id010 = id000 + vdim
                id110 = id010 + cone
                id001 = id000 + vdim2
                id101 = id001 + cone
                id011 = id001 + vdim
                id111 = id011 + cone
                rd1 = vone - rd
                gd1 = vone - gd
                bd1 = vone - bd
                w00 = rd1 * gd1
                w10 = rd * gd1
                w01 = rd1 * gd
                w11 = rd * gd
                w000 = w00 * bd1
                w100 = w10 * bd1
                w010 = w01 * bd1
                w110 = w11 * bd1
                w001 = w00 * bd
                w101 = w10 * bd
                w011 = w01 * bd
                w111 = w11 * bd

                for ch, tab in ((0, lut0), (1, lut1), (2, lut2)):
                    acc = w000 * plsc.load_gather(tab, [id000])
                    acc = acc + w100 * plsc.load_gather(tab, [id100])
                    acc = acc + w010 * plsc.load_gather(tab, [id010])
                    acc = acc + w110 * plsc.load_gather(tab, [id110])
                    acc = acc + w001 * plsc.load_gather(tab, [id001])
                    acc = acc + w101 * plsc.load_gather(tab, [id101])
                    acc = acc + w011 * plsc.load_gather(tab, [id011])
                    acc = acc + w111 * plsc.load_gather(tab, [id111])
                    out_v[ch][sl] = acc
                return 0

            lax.fori_loop(0, CHUNK // LANES, pix_body, 0)

        bufs = ((in0, sin0, out0, sout0), (in1, sin1, out1, sout1))

        for cp in in_copies(0, in0, sin0):
            cp.start()

        def pair_body(g, _):
            for b in (0, 1):
                in_b, sin_b, out_b, sout_b = bufs[b]
                in_n, sin_n, _, _ = bufs[1 - b]
                k = g * 2 + b

                @pl.when(k + 1 < n_chunks)
                def _():
                    for cp in in_copies(k + 1, in_n, sin_n):
                        cp.start()

                for cp in in_copies(k, in_b, sin_b):
                    cp.wait()

                @pl.when(k >= 2)
                def _():
                    for cp in out_copies(k - 2, out_b, sout_b):
                        cp.wait()

                compute(in_b, out_b)
                for cp in out_copies(k, out_b, sout_b):
                    cp.start()
            return 0

        lax.fori_loop(0, n_chunks // 2, pair_body, 0)

        for cp in out_copies(n_chunks - 2, out0, sout0):
            cp.wait()
        for cp in out_copies(n_chunks - 1, out1, sout1):
            cp.wait()

    return call


def kernel(LUT, x):
    B, C, H, W = x.shape
    n_pix = H * W
    xr = x.reshape(B * C, n_pix)
    lut_flat = LUT.reshape(3, DIM * DIM * DIM)
    call = _make_sc_call(B * C, n_pix)
    out = call(lut_flat[0], lut_flat[1], lut_flat[2], xr)
    return out.reshape(B, C, H, W)
